# baseline (speedup 1.0000x reference)
import jax
import jax.numpy as jnp
from jax import lax
from jax.experimental import pallas as pl
from jax.experimental.pallas import tpu as pltpu

_BBLK = 8
_DBLK = 1152
_NEG_FACTOR = 3.0


def _pass1(dim_d, nd, pred_ref, conf_ref, loc_ref, pos_ref, bg_ref, n_ref,
           locl_ref, posl_ref):
    t = pl.program_id(0)
    j = t % nd
    valid = jnp.minimum(dim_d - j * _DBLK, _DBLK)

    posf = pos_ref[...]
    lane = lax.broadcasted_iota(jnp.int32, (_BBLK, _DBLK), 1)
    posf = jnp.where(lane < valid, posf, 0.0)

    row_iota = lax.broadcasted_iota(jnp.int32, (_DBLK, 1), 0)
    rmask = row_iota < valid

    rowconf_cols = []
    bg_cols = []
    sl1_cols = []
    for b in range(_BBLK):
        x = jnp.where(rmask, pred_ref[b, :, 4:], 0.0)
        g = jnp.where(rmask, conf_ref[b, :, :], 0.0)

        m = jnp.max(x, axis=1, keepdims=True)
        se = jnp.sum(jnp.exp(x - m), axis=1, keepdims=True)
        lse = m + jnp.log(se)
        dot = jnp.sum(g * x, axis=1, keepdims=True)
        gs = jnp.sum(g, axis=1, keepdims=True)
        rowconf_cols.append(gs * lse - dot)
        bg_cols.append(g[:, -1:] * (lse - x[:, -1:]))

        d = pred_ref[b, :, :4] - loc_ref[b, :, :]
        ad = jnp.abs(d)
        sl1 = jnp.where(ad < 1.0, 0.5 * d * d, ad - 0.5)
        sl1 = jnp.where(rmask, sl1, 0.0)
        sl1_cols.append(jnp.sum(sl1, axis=1, keepdims=True))

    rowconf = jnp.concatenate(rowconf_cols, axis=1).T
    bg = jnp.concatenate(bg_cols, axis=1).T
    sl1r = jnp.concatenate(sl1_cols, axis=1).T

    bg_ref[...] = jnp.where(posf > 0.0, -jnp.inf, bg)

    n_blk = jnp.sum(posf)
    pos_loss_blk = jnp.sum(posf * rowconf)
    loc_blk = jnp.sum(posf * sl1r)

    @pl.when(t == 0)
    def _():
        n_ref[0, 0] = 0.0
        locl_ref[0, 0] = 0.0
        posl_ref[0, 0] = 0.0

    n_ref[0, 0] += n_blk
    locl_ref[0, 0] += loc_blk
    posl_ref[0, 0] += pos_loss_blk


def _monotone_key(i32):
    return i32 ^ (lax.shift_right_arithmetic(i32, 31) & jnp.int32(0x7FFFFFFF))


def _pass2(total, bg_ref, n_ref, locl_ref, posl_ref, conf_out, loc_out):
    n = n_ref[0, 0]
    posl = posl_ref[0, 0]
    loc_out[0, 0] = locl_ref[0, 0] / n

    neg_total_f = jnp.float32(total) - n
    k_f = jnp.minimum(n * _NEG_FACTOR, neg_total_f)
    k = k_f.astype(jnp.int32)
    neg_total = neg_total_f.astype(jnp.int32)

    bg = bg_ref[...]
    finite = bg != -jnp.inf
    sum_all_neg = jnp.sum(jnp.where(finite, bg, 0.0))

    @pl.when(k == neg_total)
    def _():
        conf_out[0, 0] = (posl + sum_all_neg) / n

    @pl.when(k != neg_total)
    def _():
        key = _monotone_key(lax.bitcast_convert_type(bg, jnp.int32))
        ub = key ^ jnp.int32(-2147483648)

        def bit_step(jj, carry):
            prefix, krem = carry
            b = jnp.int32(31) - jj
            cand = prefix | lax.shift_left(jnp.int32(1), b)
            match = lax.shift_right_logical(ub, b) == lax.shift_right_logical(
                cand, b)
            c1 = jnp.sum(match.astype(jnp.int32))
            take = krem <= c1
            prefix = jnp.where(take, cand, prefix)
            krem = jnp.where(take, krem, krem - c1)
            return prefix, krem

        prefix, _ = lax.fori_loop(0, 32, bit_step,
                                  (jnp.int32(0), k), unroll=True)
        t_key = prefix ^ jnp.int32(-2147483648)
        t_f = lax.bitcast_convert_type(_monotone_key(t_key), jnp.float32)
        above = key > t_key
        count_gt = jnp.sum(above.astype(jnp.int32))
        sum_gt = jnp.sum(jnp.where(above, bg, 0.0))
        neg_sum = jnp.where(
            k > 0, sum_gt + (k - count_gt).astype(jnp.float32) * t_f, 0.0)
        conf_out[0, 0] = (posl + neg_sum) / n


def kernel(predicts, pos_indicator, gt_loc, gt_conf):
    B, D, CL = predicts.shape
    C = gt_conf.shape[-1]
    M = B * D
    nb = B // _BBLK
    nd = (D + _DBLK - 1) // _DBLK

    posf = pos_indicator.astype(jnp.float32)

    smem_acc = pl.BlockSpec((1, 1), lambda t: (0, 0),
                            memory_space=pltpu.SMEM)
    bg, n_s, locl_s, posl_s = pl.pallas_call(
        lambda *refs: _pass1(D, nd, *refs),
        grid=(nb * nd,),
        in_specs=[
            pl.BlockSpec((_BBLK, _DBLK, CL), lambda t: (t // nd, t % nd, 0)),
            pl.BlockSpec((_BBLK, _DBLK, C), lambda t: (t // nd, t % nd, 0)),
            pl.BlockSpec((_BBLK, _DBLK, 4), lambda t: (t // nd, t % nd, 0)),
            pl.BlockSpec((_BBLK, _DBLK), lambda t: (t // nd, t % nd)),
        ],
        out_specs=[
            pl.BlockSpec((_BBLK, _DBLK), lambda t: (t // nd, t % nd)),
            smem_acc, smem_acc, smem_acc,
        ],
        out_shape=[
            jax.ShapeDtypeStruct((B, D), jnp.float32),
            jax.ShapeDtypeStruct((1, 1), jnp.float32),
            jax.ShapeDtypeStruct((1, 1), jnp.float32),
            jax.ShapeDtypeStruct((1, 1), jnp.float32),
        ],
    )(predicts, gt_conf, gt_loc, posf)

    smem_in = pl.BlockSpec(memory_space=pltpu.SMEM)
    conf_s, locl_o = pl.pallas_call(
        lambda *refs: _pass2(M, *refs),
        in_specs=[pl.BlockSpec(memory_space=pltpu.VMEM),
                  smem_in, smem_in, smem_in],
        out_specs=[pl.BlockSpec(memory_space=pltpu.SMEM),
                   pl.BlockSpec(memory_space=pltpu.SMEM)],
        out_shape=[
            jax.ShapeDtypeStruct((1, 1), jnp.float32),
            jax.ShapeDtypeStruct((1, 1), jnp.float32),
        ],
    )(bg, n_s, locl_s, posl_s)

    return (conf_s[0, 0], locl_o[0, 0])


# speedup vs baseline: 1.0026x; 1.0026x over previous
import jax
import jax.numpy as jnp
from jax import lax
from jax.experimental import pallas as pl
from jax.experimental.pallas import tpu as pltpu

_BBLK = 8
_DBLK = 1152
_NEG_FACTOR = 3.0


def _pass1(dim_d, nd, pred_ref, conf_ref, loc_ref, pos_ref, bg_ref,
           stats_ref):
    t = pl.program_id(0)
    j = t % nd
    valid = jnp.minimum(dim_d - j * _DBLK, _DBLK)

    posf = pos_ref[...]
    lane = lax.broadcasted_iota(jnp.int32, (_BBLK, _DBLK), 1)
    posf = jnp.where(lane < valid, posf, 0.0)

    row_iota = lax.broadcasted_iota(jnp.int32, (_DBLK, 1), 0)
    rmask = row_iota < valid

    rowconf_cols = []
    bg_cols = []
    sl1_cols = []
    for b in range(_BBLK):
        x = jnp.where(rmask, pred_ref[b, :, 4:], 0.0)
        g = jnp.where(rmask, conf_ref[b, :, :], 0.0)

        m = jnp.max(x, axis=1, keepdims=True)
        se = jnp.sum(jnp.exp(x - m), axis=1, keepdims=True)
        lse = m + jnp.log(se)
        dot = jnp.sum(g * x, axis=1, keepdims=True)
        gs = jnp.sum(g, axis=1, keepdims=True)
        rowconf_cols.append(gs * lse - dot)
        bg_cols.append(g[:, -1:] * (lse - x[:, -1:]))

        d = pred_ref[b, :, :4] - loc_ref[b, :, :]
        ad = jnp.abs(d)
        sl1 = jnp.where(ad < 1.0, 0.5 * d * d, ad - 0.5)
        sl1 = jnp.where(rmask, sl1, 0.0)
        sl1_cols.append(jnp.sum(sl1, axis=1, keepdims=True))

    rowconf = jnp.concatenate(rowconf_cols, axis=1).T
    bg = jnp.concatenate(bg_cols, axis=1).T
    sl1r = jnp.concatenate(sl1_cols, axis=1).T

    bg_ref[...] = jnp.where(posf > 0.0, -jnp.inf, bg)

    n_blk = jnp.sum(posf)
    pos_loss_blk = jnp.sum(posf * rowconf)
    loc_blk = jnp.sum(posf * sl1r)

    li = lax.broadcasted_iota(jnp.int32, (1, 1, 128), 2)
    stats = jnp.where(li == 0, n_blk,
                      jnp.where(li == 1, loc_blk,
                                jnp.where(li == 2, pos_loss_blk, 0.0)))
    stats_ref[...] = stats


def _monotone_key(i32):
    return i32 ^ (lax.shift_right_arithmetic(i32, 31) & jnp.int32(0x7FFFFFFF))


def _pass2(total, bg_ref, stats_ref, conf_out, loc_out):
    st = stats_ref[...][:, 0, :]
    n = jnp.sum(st[:, 0:1])
    posl = jnp.sum(st[:, 2:3])
    loc_out[0, 0] = jnp.sum(st[:, 1:2]) / n

    neg_total_f = jnp.float32(total) - n
    k_f = jnp.minimum(n * _NEG_FACTOR, neg_total_f)
    k = k_f.astype(jnp.int32)
    neg_total = neg_total_f.astype(jnp.int32)

    bg = bg_ref[...]
    finite = bg != -jnp.inf
    sum_all_neg = jnp.sum(jnp.where(finite, bg, 0.0))

    @pl.when(k == neg_total)
    def _():
        conf_out[0, 0] = (posl + sum_all_neg) / n

    @pl.when(k != neg_total)
    def _():
        key = _monotone_key(lax.bitcast_convert_type(bg, jnp.int32))
        ub = key ^ jnp.int32(-2147483648)

        def bit_step(jj, carry):
            prefix, krem = carry
            b = jnp.int32(31) - jj
            cand = prefix | lax.shift_left(jnp.int32(1), b)
            match = lax.shift_right_logical(ub, b) == lax.shift_right_logical(
                cand, b)
            c1 = jnp.sum(match.astype(jnp.int32))
            take = krem <= c1
            prefix = jnp.where(take, cand, prefix)
            krem = jnp.where(take, krem, krem - c1)
            return prefix, krem

        prefix, _ = lax.fori_loop(0, 32, bit_step,
                                  (jnp.int32(0), k), unroll=True)
        t_key = prefix ^ jnp.int32(-2147483648)
        t_f = lax.bitcast_convert_type(_monotone_key(t_key), jnp.float32)
        above = key > t_key
        count_gt = jnp.sum(above.astype(jnp.int32))
        sum_gt = jnp.sum(jnp.where(above, bg, 0.0))
        neg_sum = jnp.where(
            k > 0, sum_gt + (k - count_gt).astype(jnp.float32) * t_f, 0.0)
        conf_out[0, 0] = (posl + neg_sum) / n


def kernel(predicts, pos_indicator, gt_loc, gt_conf):
    B, D, CL = predicts.shape
    C = gt_conf.shape[-1]
    M = B * D
    nb = B // _BBLK
    nd = (D + _DBLK - 1) // _DBLK

    posf = pos_indicator.astype(jnp.float32)

    bg, stats = pl.pallas_call(
        lambda *refs: _pass1(D, nd, *refs),
        grid=(nb * nd,),
        in_specs=[
            pl.BlockSpec((_BBLK, _DBLK, CL), lambda t: (t // nd, t % nd, 0)),
            pl.BlockSpec((_BBLK, _DBLK, C), lambda t: (t // nd, t % nd, 0)),
            pl.BlockSpec((_BBLK, _DBLK, 4), lambda t: (t // nd, t % nd, 0)),
            pl.BlockSpec((_BBLK, _DBLK), lambda t: (t // nd, t % nd)),
        ],
        out_specs=[
            pl.BlockSpec((_BBLK, _DBLK), lambda t: (t // nd, t % nd)),
            pl.BlockSpec((1, 1, 128), lambda t: (t, 0, 0)),
        ],
        out_shape=[
            jax.ShapeDtypeStruct((B, D), jnp.float32),
            jax.ShapeDtypeStruct((nb * nd, 1, 128), jnp.float32),
        ],
    )(predicts, gt_conf, gt_loc, posf)

    conf_s, locl_o = pl.pallas_call(
        lambda *refs: _pass2(M, *refs),
        in_specs=[pl.BlockSpec(memory_space=pltpu.VMEM),
                  pl.BlockSpec(memory_space=pltpu.VMEM)],
        out_specs=[pl.BlockSpec(memory_space=pltpu.SMEM),
                   pl.BlockSpec(memory_space=pltpu.SMEM)],
        out_shape=[
            jax.ShapeDtypeStruct((1, 1), jnp.float32),
            jax.ShapeDtypeStruct((1, 1), jnp.float32),
        ],
    )(bg, stats)

    return (conf_s[0, 0], locl_o[0, 0])


# speedup vs baseline: 9.3361x; 9.3118x over previous
import jax
import jax.numpy as jnp
from jax import lax
from jax.experimental import pallas as pl
from jax.experimental.pallas import tpu as pltpu

_DCH = 1024
_NEG_FACTOR = 3.0


def _pass1(dim_d, nd, nc, pred_ref, conf_ref, loc_ref, pos_ref, bg_ref,
           stats_ref):
    j = pl.program_id(1)
    valid = dim_d - j * _DCH

    lmask = lax.broadcasted_iota(jnp.int32, (8, _DCH), 1) < valid
    posf = jnp.where(lmask, pos_ref[...], 0.0)

    m = pred_ref[4, :, :]
    for c in range(1, nc):
        m = jnp.maximum(m, pred_ref[4 + c, :, :])

    se = jnp.zeros((8, _DCH), jnp.float32)
    dot = jnp.zeros((8, _DCH), jnp.float32)
    gs = jnp.zeros((8, _DCH), jnp.float32)
    for c in range(nc):
        xc = pred_ref[4 + c, :, :]
        gc = conf_ref[c, :, :]
        se = se + jnp.exp(xc - m)
        dot = dot + gc * xc
        gs = gs + gc

    lse = m + jnp.log(se)
    rowconf = jnp.where(lmask, gs * lse - dot, 0.0)

    bgv = conf_ref[nc - 1, :, :] * (lse - pred_ref[4 + nc - 1, :, :])
    bg_ref[...] = jnp.where(posf > 0.0, -jnp.inf, bgv)

    sl1r = jnp.zeros((8, _DCH), jnp.float32)
    for c in range(4):
        d = pred_ref[c, :, :] - loc_ref[c, :, :]
        ad = jnp.abs(d)
        sl1r = sl1r + jnp.where(ad < 1.0, 0.5 * d * d, ad - 0.5)
    sl1r = jnp.where(lmask, sl1r, 0.0)

    n_blk = jnp.sum(posf)
    pos_loss_blk = jnp.sum(posf * rowconf)
    loc_blk = jnp.sum(posf * sl1r)

    li = lax.broadcasted_iota(jnp.int32, (1, 1, 128), 2)
    stats = jnp.where(li == 0, n_blk,
                      jnp.where(li == 1, loc_blk,
                                jnp.where(li == 2, pos_loss_blk, 0.0)))
    stats_ref[...] = stats


def _monotone_key(i32):
    return i32 ^ (lax.shift_right_arithmetic(i32, 31) & jnp.int32(0x7FFFFFFF))


def _pass2(total, bg_ref, stats_ref, conf_out, loc_out):
    st = stats_ref[...][:, 0, :]
    n = jnp.sum(st[:, 0:1])
    posl = jnp.sum(st[:, 2:3])
    loc_out[0, 0] = jnp.sum(st[:, 1:2]) / n

    neg_total_f = jnp.float32(total) - n
    k_f = jnp.minimum(n * _NEG_FACTOR, neg_total_f)
    k = k_f.astype(jnp.int32)
    neg_total = neg_total_f.astype(jnp.int32)

    bg = bg_ref[...]
    finite = bg != -jnp.inf
    sum_all_neg = jnp.sum(jnp.where(finite, bg, 0.0))

    @pl.when(k == neg_total)
    def _():
        conf_out[0, 0] = (posl + sum_all_neg) / n

    @pl.when(k != neg_total)
    def _():
        key = _monotone_key(lax.bitcast_convert_type(bg, jnp.int32))
        ub = key ^ jnp.int32(-2147483648)

        def bit_step(jj, carry):
            prefix, krem = carry
            b = jnp.int32(31) - jj
            cand = prefix | lax.shift_left(jnp.int32(1), b)
            match = lax.shift_right_logical(ub, b) == lax.shift_right_logical(
                cand, b)
            c1 = jnp.sum(match.astype(jnp.int32))
            take = krem <= c1
            prefix = jnp.where(take, cand, prefix)
            krem = jnp.where(take, krem, krem - c1)
            return prefix, krem

        prefix, _ = lax.fori_loop(0, 32, bit_step,
                                  (jnp.int32(0), k), unroll=True)
        t_key = prefix ^ jnp.int32(-2147483648)
        t_f = lax.bitcast_convert_type(_monotone_key(t_key), jnp.float32)
        above = key > t_key
        count_gt = jnp.sum(above.astype(jnp.int32))
        sum_gt = jnp.sum(jnp.where(above, bg, 0.0))
        neg_sum = jnp.where(
            k > 0, sum_gt + (k - count_gt).astype(jnp.float32) * t_f, 0.0)
        conf_out[0, 0] = (posl + neg_sum) / n


def kernel(predicts, pos_indicator, gt_loc, gt_conf):
    B, D, CL = predicts.shape
    C = gt_conf.shape[-1]
    M = B * D
    nb = B // 8
    nd = (D + _DCH - 1) // _DCH

    pred_t = jnp.transpose(predicts, (2, 0, 1))
    conf_t = jnp.transpose(gt_conf, (2, 0, 1))
    loc_t = jnp.transpose(gt_loc, (2, 0, 1))
    posf = pos_indicator.astype(jnp.float32)

    bg, stats = pl.pallas_call(
        lambda *refs: _pass1(D, nd, C, *refs),
        grid=(nb, nd),
        in_specs=[
            pl.BlockSpec((CL, 8, _DCH), lambda i, j: (0, i, j)),
            pl.BlockSpec((C, 8, _DCH), lambda i, j: (0, i, j)),
            pl.BlockSpec((4, 8, _DCH), lambda i, j: (0, i, j)),
            pl.BlockSpec((8, _DCH), lambda i, j: (i, j)),
        ],
        out_specs=[
            pl.BlockSpec((8, _DCH), lambda i, j: (i, j)),
            pl.BlockSpec((1, 1, 128), lambda i, j: (i * nd + j, 0, 0)),
        ],
        out_shape=[
            jax.ShapeDtypeStruct((B, D), jnp.float32),
            jax.ShapeDtypeStruct((nb * nd, 1, 128), jnp.float32),
        ],
    )(pred_t, conf_t, loc_t, posf)

    conf_s, locl_o = pl.pallas_call(
        lambda *refs: _pass2(M, *refs),
        in_specs=[pl.BlockSpec(memory_space=pltpu.VMEM),
                  pl.BlockSpec(memory_space=pltpu.VMEM)],
        out_specs=[pl.BlockSpec(memory_space=pltpu.SMEM),
                   pl.BlockSpec(memory_space=pltpu.SMEM)],
        out_shape=[
            jax.ShapeDtypeStruct((1, 1), jnp.float32),
            jax.ShapeDtypeStruct((1, 1), jnp.float32),
        ],
    )(bg, stats)

    return (conf_s[0, 0], locl_o[0, 0])


# speedup vs baseline: 10.0631x; 1.0779x over previous
import jax
import jax.numpy as jnp
from jax import lax
from jax.experimental import pallas as pl
from jax.experimental.pallas import tpu as pltpu

_DCH = 2944
_NEG_FACTOR = 3.0


def _pass1(dim_d, nd, nc, pred_ref, conf_ref, loc_ref, pos_ref, bg_ref,
           stats_ref):
    j = pl.program_id(1)
    valid = dim_d - j * _DCH

    lmask = lax.broadcasted_iota(jnp.int32, (8, _DCH), 1) < valid
    posf = jnp.where(lmask, pos_ref[...], 0.0)

    m = pred_ref[4, :, :]
    for c in range(1, nc):
        m = jnp.maximum(m, pred_ref[4 + c, :, :])

    se = jnp.zeros((8, _DCH), jnp.float32)
    dot = jnp.zeros((8, _DCH), jnp.float32)
    gs = jnp.zeros((8, _DCH), jnp.float32)
    for c in range(nc):
        xc = pred_ref[4 + c, :, :]
        gc = conf_ref[c, :, :]
        se = se + jnp.exp(xc - m)
        dot = dot + gc * xc
        gs = gs + gc

    lse = m + jnp.log(se)
    rowconf = jnp.where(lmask, gs * lse - dot, 0.0)

    bgv = conf_ref[nc - 1, :, :] * (lse - pred_ref[4 + nc - 1, :, :])
    bg_ref[...] = jnp.where(posf > 0.0, -jnp.inf, bgv)

    sl1r = jnp.zeros((8, _DCH), jnp.float32)
    for c in range(4):
        d = pred_ref[c, :, :] - loc_ref[c, :, :]
        ad = jnp.abs(d)
        sl1r = sl1r + jnp.where(ad < 1.0, 0.5 * d * d, ad - 0.5)
    sl1r = jnp.where(lmask, sl1r, 0.0)

    n_blk = jnp.sum(posf)
    pos_loss_blk = jnp.sum(posf * rowconf)
    loc_blk = jnp.sum(posf * sl1r)

    li = lax.broadcasted_iota(jnp.int32, (1, 1, 128), 2)
    stats = jnp.where(li == 0, n_blk,
                      jnp.where(li == 1, loc_blk,
                                jnp.where(li == 2, pos_loss_blk, 0.0)))
    stats_ref[...] = stats


def _monotone_key(i32):
    return i32 ^ (lax.shift_right_arithmetic(i32, 31) & jnp.int32(0x7FFFFFFF))


def _pass2(total, bg_ref, stats_ref, conf_out, loc_out):
    st = stats_ref[...][:, 0, :]
    n = jnp.sum(st[:, 0:1])
    posl = jnp.sum(st[:, 2:3])
    loc_out[0, 0] = jnp.sum(st[:, 1:2]) / n

    neg_total_f = jnp.float32(total) - n
    k_f = jnp.minimum(n * _NEG_FACTOR, neg_total_f)
    k = k_f.astype(jnp.int32)
    neg_total = neg_total_f.astype(jnp.int32)

    bg = bg_ref[...]
    finite = bg != -jnp.inf
    sum_all_neg = jnp.sum(jnp.where(finite, bg, 0.0))

    @pl.when(k == neg_total)
    def _():
        conf_out[0, 0] = (posl + sum_all_neg) / n

    @pl.when(k != neg_total)
    def _():
        key = _monotone_key(lax.bitcast_convert_type(bg, jnp.int32))
        ub = key ^ jnp.int32(-2147483648)

        def bit_step(jj, carry):
            prefix, krem = carry
            b = jnp.int32(31) - jj
            cand = prefix | lax.shift_left(jnp.int32(1), b)
            match = lax.shift_right_logical(ub, b) == lax.shift_right_logical(
                cand, b)
            c1 = jnp.sum(match.astype(jnp.int32))
            take = krem <= c1
            prefix = jnp.where(take, cand, prefix)
            krem = jnp.where(take, krem, krem - c1)
            return prefix, krem

        prefix, _ = lax.fori_loop(0, 32, bit_step,
                                  (jnp.int32(0), k), unroll=True)
        t_key = prefix ^ jnp.int32(-2147483648)
        t_f = lax.bitcast_convert_type(_monotone_key(t_key), jnp.float32)
        above = key > t_key
        count_gt = jnp.sum(above.astype(jnp.int32))
        sum_gt = jnp.sum(jnp.where(above, bg, 0.0))
        neg_sum = jnp.where(
            k > 0, sum_gt + (k - count_gt).astype(jnp.float32) * t_f, 0.0)
        conf_out[0, 0] = (posl + neg_sum) / n


def kernel(predicts, pos_indicator, gt_loc, gt_conf):
    B, D, CL = predicts.shape
    C = gt_conf.shape[-1]
    M = B * D
    nb = B // 8
    nd = (D + _DCH - 1) // _DCH

    pred_t = jnp.transpose(predicts, (2, 0, 1))
    conf_t = jnp.transpose(gt_conf, (2, 0, 1))
    loc_t = jnp.transpose(gt_loc, (2, 0, 1))
    posf = pos_indicator.astype(jnp.float32)

    bg, stats = pl.pallas_call(
        lambda *refs: _pass1(D, nd, C, *refs),
        grid=(nb, nd),
        in_specs=[
            pl.BlockSpec((CL, 8, _DCH), lambda i, j: (0, i, j)),
            pl.BlockSpec((C, 8, _DCH), lambda i, j: (0, i, j)),
            pl.BlockSpec((4, 8, _DCH), lambda i, j: (0, i, j)),
            pl.BlockSpec((8, _DCH), lambda i, j: (i, j)),
        ],
        out_specs=[
            pl.BlockSpec((8, _DCH), lambda i, j: (i, j)),
            pl.BlockSpec((1, 1, 128), lambda i, j: (i * nd + j, 0, 0)),
        ],
        out_shape=[
            jax.ShapeDtypeStruct((B, D), jnp.float32),
            jax.ShapeDtypeStruct((nb * nd, 1, 128), jnp.float32),
        ],
    )(pred_t, conf_t, loc_t, posf)

    conf_s, locl_o = pl.pallas_call(
        lambda *refs: _pass2(M, *refs),
        in_specs=[pl.BlockSpec(memory_space=pltpu.VMEM),
                  pl.BlockSpec(memory_space=pltpu.VMEM)],
        out_specs=[pl.BlockSpec(memory_space=pltpu.SMEM),
                   pl.BlockSpec(memory_space=pltpu.SMEM)],
        out_shape=[
            jax.ShapeDtypeStruct((1, 1), jnp.float32),
            jax.ShapeDtypeStruct((1, 1), jnp.float32),
        ],
    )(bg, stats)

    return (conf_s[0, 0], locl_o[0, 0])
